# baseline (speedup 1.0000x reference)
import functools

import jax
import jax.numpy as jnp
from jax import lax
from jax.experimental import pallas as pl
from jax.experimental.pallas import tpu as pltpu
from jax.experimental.pallas import tpu_sc as plsc

N_NODES = 10000
N_EDGES = 160000
N_TGT = 50000
IN_D = 128
HID_D = 16
ATTR_D = 16
NNH = 100
NNH_P = 104
WID = NNH_P * 16

NC, NS = 2, 16
NW = NC * NS
CH = 128

E_PAD = 163840
T_PAD = 53248
T2_PAD = 114688
N_PAD = 10112
TBLK = 4096
TNB = T_PAD // TBLK


def _vmesh():
    return plsc.VectorSubcoreMesh(core_axis_name="c", subcore_axis_name="s")


def _sc_gather(table, idx3d, out_rows, n_ch):
    per_w = out_rows // NW
    NB = 4

    @functools.partial(
        pl.kernel,
        out_type=jax.ShapeDtypeStruct((out_rows, IN_D), jnp.float32),
        mesh=_vmesh(),
        scratch_types=[
            pltpu.VMEM((n_ch, CH), jnp.int32),
            pltpu.VMEM((NB, CH, IN_D), jnp.float32),
            pltpu.SemaphoreType.DMA,
            pltpu.SemaphoreType.DMA,
            pltpu.SemaphoreType.DMA,
            pltpu.SemaphoreType.DMA,
            pltpu.SemaphoreType.DMA,
            pltpu.SemaphoreType.DMA,
            pltpu.SemaphoreType.DMA,
            pltpu.SemaphoreType.DMA,
        ],
    )
    def k(table_hbm, idx_hbm, out_hbm, idx_v, rows_v, *sems):
        gsem = sems[:NB]
        wsem = sems[NB:]
        wid = lax.axis_index("s") * NC + lax.axis_index("c")
        base = wid * per_w
        pltpu.sync_copy(idx_hbm.at[wid], idx_v)

        def g_desc(jj, b):
            return pltpu.make_async_copy(
                table_hbm.at[idx_v.at[jj]], rows_v.at[b], gsem[b])

        def w_desc(jj, b):
            return pltpu.make_async_copy(
                rows_v.at[b], out_hbm.at[pl.ds(base + jj * CH, CH)], wsem[b])

        for b in range(NB):
            g_desc(b, b).start()

        @pl.loop(0, n_ch, step=NB)
        def _(j):
            for b in range(NB):
                g_desc(j + b, b).wait()
                w_desc(j + b, b).start()
            for b in range(NB):
                w_desc(j + b, b).wait()

                @pl.when(j + NB + b < n_ch)
                def _():
                    g_desc(j + NB + b, b).start()

    return k(table, idx3d)


def _sc_scatter_add(msg, dst3d, zero_init):
    rows_per_tile = N_PAD // NS
    e_half = E_PAD // NC
    per_w = e_half // NS
    n_ch = per_w // CH
    NB = 2

    @functools.partial(
        pl.kernel,
        out_type=jax.ShapeDtypeStruct((NC, N_PAD, IN_D), jnp.float32),
        mesh=_vmesh(),
        scratch_types=[
            pltpu.VMEM((n_ch, CH), jnp.int32),
            pltpu.VMEM((NB, CH, IN_D), jnp.float32),
            pltpu.VMEM_SHARED((N_PAD, IN_D), jnp.float32),
            pltpu.SemaphoreType.DMA,
            pltpu.SemaphoreType.DMA,
            pltpu.SemaphoreType.DMA,
            pltpu.SemaphoreType.DMA,
        ],
    )
    def k(msg_hbm, dst_hbm, zero_hbm, out_hbm, idx_v, val_v, agg_sh, *sems):
        lsem = sems[:NB]
        ssem = sems[NB:]
        c = lax.axis_index("c")
        s = lax.axis_index("s")
        wid = s * NC + c
        r0 = s * rows_per_tile
        pltpu.sync_copy(dst_hbm.at[wid], idx_v)
        pltpu.sync_copy(zero_hbm.at[pl.ds(r0, rows_per_tile)],
                        agg_sh.at[pl.ds(r0, rows_per_tile)])
        plsc.subcore_barrier()
        base = wid * per_w

        def l_desc(jj, b):
            return pltpu.make_async_copy(
                msg_hbm.at[pl.ds(base + jj * CH, CH)], val_v.at[b], lsem[b])

        def s_desc(jj, b):
            return pltpu.make_async_copy(
                val_v.at[b], agg_sh.at[idx_v.at[jj]], ssem[b])

        for b in range(NB):
            l_desc(b, b).start()

        @pl.loop(0, n_ch, step=NB)
        def _(j):
            for b in range(NB):
                l_desc(j + b, b).wait()
                s_desc(j + b, b).start(add=True)
            for b in range(NB):
                s_desc(j + b, b).wait()

                @pl.when(j + NB + b < n_ch)
                def _():
                    l_desc(j + NB + b, b).start()

        plsc.subcore_barrier()
        pltpu.sync_copy(agg_sh.at[pl.ds(r0, rows_per_tile)],
                        out_hbm.at[c, pl.ds(r0, rows_per_tile)])

    return k(msg, dst3d, zero_init)


_DN = (((1,), (0,)), ((), ()))


def _edge_messages(feat, ea, w1b, b1b, w2p, s_sel, b2r, blk):
    def body(feat_ref, ea_ref, w1b_ref, b1b_ref, w2p_ref, s_ref, b2r_ref, out_ref):
        ea_b = ea_ref[...].astype(jnp.bfloat16)
        pre = lax.dot_general(ea_b, w1b_ref[...], _DN,
                              preferred_element_type=jnp.float32) + b1b_ref[...]
        h2 = jnp.maximum(pre, 0.0)
        xb = feat_ref[...].astype(jnp.bfloat16)
        g = lax.dot_general(xb, w2p_ref[...], _DN,
                            preferred_element_type=jnp.float32)
        p = (h2 * g).astype(jnp.bfloat16)
        m = lax.dot_general(p, s_ref[...], _DN, preferred_element_type=jnp.float32)
        m = m + lax.dot_general(xb, b2r_ref[...], _DN,
                                preferred_element_type=jnp.float32)
        out_ref[...] = jnp.concatenate(
            [m, jnp.zeros((m.shape[0], IN_D - HID_D), jnp.float32)], axis=1)

    return pl.pallas_call(
        body,
        grid=(E_PAD // blk,),
        in_specs=[
            pl.BlockSpec((blk, IN_D), lambda i: (i, 0)),
            pl.BlockSpec((blk, ATTR_D), lambda i: (i, 0)),
            pl.BlockSpec((ATTR_D, WID), lambda i: (0, 0)),
            pl.BlockSpec((1, WID), lambda i: (0, 0)),
            pl.BlockSpec((IN_D, WID), lambda i: (0, 0)),
            pl.BlockSpec((WID, HID_D), lambda i: (0, 0)),
            pl.BlockSpec((IN_D, HID_D), lambda i: (0, 0)),
        ],
        out_specs=pl.BlockSpec((blk, IN_D), lambda i: (i, 0)),
        out_shape=jax.ShapeDtypeStruct((E_PAD, IN_D), jnp.float32),
        compiler_params=pltpu.CompilerParams(
            dimension_semantics=("parallel",)),
    )(feat, ea, w1b, b1b, w2p, s_sel, b2r)


def _node_update(aggp, feat, root_pad, bias_row, relu):
    def body(agg_ref, feat_ref, root_ref, bias_ref, out_ref):
        a = (agg_ref[0, :N_NODES, :HID_D]
             + agg_ref[1, :N_NODES, :HID_D])
        fb = feat_ref[...].astype(jnp.bfloat16)
        r = lax.dot_general(fb, root_ref[...], _DN,
                            preferred_element_type=jnp.float32)
        o = a + r + bias_ref[...]
        if relu:
            o = jnp.maximum(o, 0.0)
        out_ref[...] = jnp.concatenate(
            [o, jnp.zeros((N_NODES, IN_D - HID_D), jnp.float32)], axis=1)

    return pl.pallas_call(
        body,
        in_specs=[
            pl.BlockSpec((NC, N_PAD, IN_D), lambda: (0, 0, 0)),
            pl.BlockSpec((N_NODES, IN_D), lambda: (0, 0)),
            pl.BlockSpec((IN_D, HID_D), lambda: (0, 0)),
            pl.BlockSpec((1, HID_D), lambda: (0, 0)),
        ],
        out_specs=pl.BlockSpec((N_NODES, IN_D), lambda: (0, 0)),
        out_shape=jax.ShapeDtypeStruct((N_NODES, IN_D), jnp.float32),
    )(aggp, feat, root_pad, bias_row)


def _head_stats(z):
    def body(z1_ref, z2_ref, out_ref):
        j = pl.program_id(0)

        @pl.when(j == 0)
        def _():
            out_ref[...] = jnp.zeros((8, IN_D), jnp.float32)

        rows = lax.broadcasted_iota(jnp.int32, (TBLK, 1), 0) + j * TBLK
        m = (rows < N_TGT).astype(jnp.float32)
        z1 = z1_ref[...] * m
        z2 = z2_ref[...] * m
        s1 = jnp.sum(z1, axis=0, keepdims=True)
        q1 = jnp.sum(z1 * z1, axis=0, keepdims=True)
        s2 = jnp.sum(z2, axis=0, keepdims=True)
        q2 = jnp.sum(z2 * z2, axis=0, keepdims=True)
        zero4 = jnp.zeros((4, IN_D), jnp.float32)
        upd = jnp.concatenate([s1, q1, s2, q2, zero4], axis=0)
        out_ref[...] = out_ref[...] + upd

    return pl.pallas_call(
        body,
        grid=(TNB,),
        in_specs=[
            pl.BlockSpec((TBLK, IN_D), lambda j: (j, 0)),
            pl.BlockSpec((TBLK, IN_D), lambda j: (j + TNB, 0)),
        ],
        out_specs=pl.BlockSpec((8, IN_D), lambda j: (0, 0)),
        out_shape=jax.ShapeDtypeStruct((8, IN_D), jnp.float32),
    )(z, z)


def _head_apply(z, stats, ga, gb, ba, bb, wa, wb, lb_row):
    def body(z1_ref, z2_ref, st_ref, ga_ref, gb_ref, ba_ref, bb_ref,
             wa_ref, wb_ref, lb_ref, out_ref):
        cnt = float(N_TGT)
        mean1 = st_ref[0:1, :] / cnt
        msq1 = st_ref[1:2, :] / cnt
        mean2 = st_ref[2:3, :] / cnt
        msq2 = st_ref[3:4, :] / cnt
        inv1 = lax.rsqrt(msq1 - mean1 * mean1 + 1e-5)
        inv2 = lax.rsqrt(msq2 - mean2 * mean2 + 1e-5)
        sc1 = inv1 * ga_ref[...]
        sc2 = inv2 * gb_ref[...]
        sh1 = ba_ref[...] - mean1 * sc1
        sh2 = bb_ref[...] - mean2 * sc2
        wa = wa_ref[...]
        wb = wb_ref[...]
        bias_eff = (lax.dot_general(sh1, wa, _DN, preferred_element_type=jnp.float32)
                    + lax.dot_general(sh2, wb, _DN, preferred_element_type=jnp.float32)
                    + lb_ref[...])
        x1 = (z1_ref[...] * sc1).astype(jnp.bfloat16)
        x2 = (z2_ref[...] * sc2).astype(jnp.bfloat16)
        o = lax.dot_general(x1, wa.astype(jnp.bfloat16), _DN,
                            preferred_element_type=jnp.float32)
        o = o + lax.dot_general(x2, wb.astype(jnp.bfloat16), _DN,
                                preferred_element_type=jnp.float32)
        o = o + bias_eff
        out_ref[...] = 1.0 / (1.0 + jnp.exp(-o))

    return pl.pallas_call(
        body,
        grid=(TNB,),
        in_specs=[
            pl.BlockSpec((TBLK, IN_D), lambda j: (j, 0)),
            pl.BlockSpec((TBLK, IN_D), lambda j: (j + TNB, 0)),
            pl.BlockSpec((8, IN_D), lambda j: (0, 0)),
            pl.BlockSpec((1, IN_D), lambda j: (0, 0)),
            pl.BlockSpec((1, IN_D), lambda j: (0, 0)),
            pl.BlockSpec((1, IN_D), lambda j: (0, 0)),
            pl.BlockSpec((1, IN_D), lambda j: (0, 0)),
            pl.BlockSpec((IN_D, HID_D), lambda j: (0, 0)),
            pl.BlockSpec((IN_D, HID_D), lambda j: (0, 0)),
            pl.BlockSpec((1, HID_D), lambda j: (0, 0)),
        ],
        out_specs=pl.BlockSpec((TBLK, HID_D), lambda j: (j, 0)),
        out_shape=jax.ShapeDtypeStruct((T_PAD, HID_D), jnp.float32),
        compiler_params=pltpu.CompilerParams(
            dimension_semantics=("arbitrary",)),
    )(z, z, stats, ga, gb, ba, bb, wa, wb, lb_row)


def _prep_edge_weights(w1, b1, w2, b2, d_in):
    w1p = jnp.pad(w1, ((0, 0), (0, NNH_P - NNH)))
    b1p = jnp.pad(b1, (0, NNH_P - NNH))
    w2p = jnp.pad(w2, ((0, NNH_P - NNH), (0, 0)))
    w1b = jnp.repeat(w1p, HID_D, axis=1).astype(jnp.bfloat16)
    b1b = jnp.repeat(b1p, HID_D).reshape(1, WID)
    w2perm = (w2p.reshape(NNH_P, d_in, HID_D)
              .transpose(1, 0, 2).reshape(d_in, WID))
    w2perm = jnp.pad(w2perm, ((0, IN_D - d_in), (0, 0))).astype(jnp.bfloat16)
    b2r = jnp.pad(b2.reshape(d_in, HID_D),
                  ((0, IN_D - d_in), (0, 0))).astype(jnp.bfloat16)
    return w1b, b1b, w2perm, b2r


def _pad_lanes(v, d=IN_D):
    return jnp.pad(v.reshape(1, -1), ((0, 0), (0, d - v.shape[-1])))


def kernel(x, edge_index, edge_attr, target_edge_index,
           nn1_W1, nn1_b1, nn1_W2, nn1_b2, root1, bias1,
           nn2_W1, nn2_b1, nn2_W2, nn2_b2, root2, bias2,
           bn_g, bn_b, lin_W, lin_b):
    src = edge_index[0]
    dst = edge_index[1]
    e_fill = E_PAD - N_EDGES
    src_p = jnp.concatenate([src, jnp.zeros((e_fill,), jnp.int32)])
    dst_p = jnp.concatenate([dst, jnp.full((e_fill,), N_NODES, jnp.int32)])
    ea_p = jnp.concatenate(
        [edge_attr, jnp.zeros((e_fill, ATTR_D), jnp.float32)])
    t_fill = jnp.zeros((T_PAD - N_TGT,), jnp.int32)
    tcat = jnp.concatenate(
        [target_edge_index[0], t_fill, target_edge_index[1], t_fill,
         jnp.zeros((T2_PAD - 2 * T_PAD,), jnp.int32)])
    src3d = src_p.reshape(NW, E_PAD // (NW * CH), CH)
    dst3d = dst_p.reshape(NW, E_PAD // (NW * CH), CH)
    t3d = tcat.reshape(NW, T2_PAD // (NW * CH), CH)
    zero_agg = jnp.zeros((N_PAD, IN_D), jnp.float32)

    w1b1, b1b1, w2p1, b2r1 = _prep_edge_weights(nn1_W1, nn1_b1, nn1_W2, nn1_b2, IN_D)
    w1b2, b1b2, w2p2, b2r2 = _prep_edge_weights(nn2_W1, nn2_b1, nn2_W2, nn2_b2, HID_D)
    s_sel = jnp.tile(jnp.eye(HID_D, dtype=jnp.bfloat16), (NNH_P, 1))
    root1b = root1.astype(jnp.bfloat16)
    root2b = jnp.pad(root2, ((0, IN_D - HID_D), (0, 0))).astype(jnp.bfloat16)

    xs = _sc_gather(x, src3d, E_PAD, E_PAD // (NW * CH))
    msg1 = _edge_messages(xs, ea_p, w1b1, b1b1, w2p1, s_sel, b2r1, blk=1024)
    agg1 = _sc_scatter_add(msg1, dst3d, zero_agg)
    h = _node_update(agg1, x, root1b, bias1.reshape(1, HID_D), relu=True)
    hs = _sc_gather(h, src3d, E_PAD, E_PAD // (NW * CH))
    msg2 = _edge_messages(hs, ea_p, w1b2, b1b2, w2p2, s_sel, b2r2, blk=1024)
    agg2 = _sc_scatter_add(msg2, dst3d, zero_agg)
    h2 = _node_update(agg2, h, root2b, bias2.reshape(1, HID_D), relu=False)
    z = _sc_gather(h2, t3d, T2_PAD, T2_PAD // (NW * CH))
    stats = _head_stats(z)
    out = _head_apply(z, stats,
                      _pad_lanes(bn_g[:HID_D]), _pad_lanes(bn_g[HID_D:]),
                      _pad_lanes(bn_b[:HID_D]), _pad_lanes(bn_b[HID_D:]),
                      jnp.pad(lin_W[:HID_D], ((0, IN_D - HID_D), (0, 0))),
                      jnp.pad(lin_W[HID_D:], ((0, IN_D - HID_D), (0, 0))),
                      lin_b.reshape(1, HID_D))
    return out[:N_TGT]


# speedup vs baseline: 1.6718x; 1.6718x over previous
import functools

import jax
import jax.numpy as jnp
from jax import lax
from jax.experimental import pallas as pl
from jax.experimental.pallas import tpu as pltpu
from jax.experimental.pallas import tpu_sc as plsc

N_NODES = 10000
N_EDGES = 160000
N_TGT = 50000
IN_D = 128
HID_D = 16
ATTR_D = 16
NNH = 100
NNH_P = 104
WID = NNH_P * 16

NC, NS = 2, 16
NW = NC * NS
CH = 128

E_PAD = 163840
T_PAD = 53248
T2_PAD = 114688
N_PAD = 10112
TBLK = 4096
TNB = T_PAD // TBLK


def _vmesh():
    return plsc.VectorSubcoreMesh(core_axis_name="c", subcore_axis_name="s")


def _sc_gather(table, idx3d, out_rows, n_ch):
    per_w = out_rows // NW
    NB = 4

    @functools.partial(
        pl.kernel,
        out_type=jax.ShapeDtypeStruct((out_rows, IN_D), jnp.float32),
        mesh=_vmesh(),
        scratch_types=[
            pltpu.VMEM((n_ch, CH), jnp.int32),
            pltpu.VMEM((NB, CH, IN_D), jnp.float32),
            pltpu.SemaphoreType.DMA,
            pltpu.SemaphoreType.DMA,
            pltpu.SemaphoreType.DMA,
            pltpu.SemaphoreType.DMA,
            pltpu.SemaphoreType.DMA,
            pltpu.SemaphoreType.DMA,
            pltpu.SemaphoreType.DMA,
            pltpu.SemaphoreType.DMA,
        ],
    )
    def k(table_hbm, idx_hbm, out_hbm, idx_v, rows_v, *sems):
        gsem = sems[:NB]
        wsem = sems[NB:]
        wid = lax.axis_index("s") * NC + lax.axis_index("c")
        base = wid * per_w
        pltpu.sync_copy(idx_hbm.at[wid], idx_v)

        def g_desc(jj, b):
            return pltpu.make_async_copy(
                table_hbm.at[idx_v.at[jj]], rows_v.at[b], gsem[b])

        def w_desc(jj, b):
            return pltpu.make_async_copy(
                rows_v.at[b], out_hbm.at[pl.ds(base + jj * CH, CH)], wsem[b])

        for b in range(NB):
            g_desc(b, b).start()

        @pl.loop(0, n_ch, step=NB)
        def _(j):
            for b in range(NB):
                g_desc(j + b, b).wait()
                w_desc(j + b, b).start()
            for b in range(NB):
                w_desc(j + b, b).wait()

                @pl.when(j + NB + b < n_ch)
                def _():
                    g_desc(j + NB + b, b).start()

    return k(table, idx3d)


def _sc_scatter_add(msg, dst3d, zero_init):
    rows_per_tile = N_PAD // NS
    e_half = E_PAD // NC
    per_w = e_half // NS
    n_ch = per_w // CH
    NB = 2

    @functools.partial(
        pl.kernel,
        out_type=jax.ShapeDtypeStruct((NC, N_PAD, IN_D), jnp.float32),
        mesh=_vmesh(),
        scratch_types=[
            pltpu.VMEM((n_ch, CH), jnp.int32),
            pltpu.VMEM((NB, CH, IN_D), jnp.float32),
            pltpu.VMEM_SHARED((N_PAD, IN_D), jnp.float32),
            pltpu.SemaphoreType.DMA,
            pltpu.SemaphoreType.DMA,
            pltpu.SemaphoreType.DMA,
            pltpu.SemaphoreType.DMA,
        ],
    )
    def k(msg_hbm, dst_hbm, zero_hbm, out_hbm, idx_v, val_v, agg_sh, *sems):
        lsem = sems[:NB]
        ssem = sems[NB:]
        c = lax.axis_index("c")
        s = lax.axis_index("s")
        wid = s * NC + c
        r0 = s * rows_per_tile
        pltpu.sync_copy(dst_hbm.at[wid], idx_v)
        pltpu.sync_copy(zero_hbm.at[pl.ds(r0, rows_per_tile)],
                        agg_sh.at[pl.ds(r0, rows_per_tile)])
        plsc.subcore_barrier()
        base = wid * per_w

        def l_desc(jj, b):
            return pltpu.make_async_copy(
                msg_hbm.at[pl.ds(base + jj * CH, CH)], val_v.at[b], lsem[b])

        def s_desc(jj, b):
            return pltpu.make_async_copy(
                val_v.at[b], agg_sh.at[idx_v.at[jj]], ssem[b])

        for b in range(NB):
            l_desc(b, b).start()

        @pl.loop(0, n_ch, step=NB)
        def _(j):
            for b in range(NB):
                l_desc(j + b, b).wait()
                s_desc(j + b, b).start(add=True)
            for b in range(NB):
                s_desc(j + b, b).wait()

                @pl.when(j + NB + b < n_ch)
                def _():
                    l_desc(j + NB + b, b).start()

        plsc.subcore_barrier()
        pltpu.sync_copy(agg_sh.at[pl.ds(r0, rows_per_tile)],
                        out_hbm.at[c, pl.ds(r0, rows_per_tile)])

    return k(msg, dst3d, zero_init)


_DN = (((1,), (0,)), ((), ()))


def _edge_messages(feat, ea, w1b, b1b, w2p, s_sel, b2r, blk):
    def body(feat_ref, ea_ref, w1b_ref, b1b_ref, w2p_ref, s_ref, b2r_ref, out_ref):
        ea_b = ea_ref[...].astype(jnp.bfloat16)
        pre = lax.dot_general(ea_b, w1b_ref[...], _DN,
                              preferred_element_type=jnp.float32) + b1b_ref[...]
        h2 = jnp.maximum(pre, 0.0)
        xb = feat_ref[...].astype(jnp.bfloat16)
        g = lax.dot_general(xb, w2p_ref[...], _DN,
                            preferred_element_type=jnp.float32)
        p = (h2 * g).astype(jnp.bfloat16)
        m = lax.dot_general(p, s_ref[...], _DN, preferred_element_type=jnp.float32)
        m = m + lax.dot_general(xb, b2r_ref[...], _DN,
                                preferred_element_type=jnp.float32)
        out_ref[...] = jnp.concatenate(
            [m, jnp.zeros((m.shape[0], IN_D - HID_D), jnp.float32)], axis=1)

    return pl.pallas_call(
        body,
        grid=(E_PAD // blk,),
        in_specs=[
            pl.BlockSpec((blk, IN_D), lambda i: (i, 0)),
            pl.BlockSpec((blk, ATTR_D), lambda i: (i, 0)),
            pl.BlockSpec((ATTR_D, WID), lambda i: (0, 0)),
            pl.BlockSpec((1, WID), lambda i: (0, 0)),
            pl.BlockSpec((IN_D, WID), lambda i: (0, 0)),
            pl.BlockSpec((WID, HID_D), lambda i: (0, 0)),
            pl.BlockSpec((IN_D, HID_D), lambda i: (0, 0)),
        ],
        out_specs=pl.BlockSpec((blk, IN_D), lambda i: (i, 0)),
        out_shape=jax.ShapeDtypeStruct((E_PAD, IN_D), jnp.float32),
        compiler_params=pltpu.CompilerParams(
            dimension_semantics=("parallel",)),
    )(feat, ea, w1b, b1b, w2p, s_sel, b2r)


def _node_update(aggp, feat, root_pad, bias_row, relu):
    def body(agg_ref, feat_ref, root_ref, bias_ref, out_ref):
        a = (agg_ref[0, :N_NODES, :HID_D]
             + agg_ref[1, :N_NODES, :HID_D])
        fb = feat_ref[...].astype(jnp.bfloat16)
        r = lax.dot_general(fb, root_ref[...], _DN,
                            preferred_element_type=jnp.float32)
        o = a + r + bias_ref[...]
        if relu:
            o = jnp.maximum(o, 0.0)
        out_ref[...] = jnp.concatenate(
            [o, jnp.zeros((N_NODES, IN_D - HID_D), jnp.float32)], axis=1)

    return pl.pallas_call(
        body,
        in_specs=[
            pl.BlockSpec((NC, N_PAD, IN_D), lambda: (0, 0, 0)),
            pl.BlockSpec((N_NODES, IN_D), lambda: (0, 0)),
            pl.BlockSpec((IN_D, HID_D), lambda: (0, 0)),
            pl.BlockSpec((1, HID_D), lambda: (0, 0)),
        ],
        out_specs=pl.BlockSpec((N_NODES, IN_D), lambda: (0, 0)),
        out_shape=jax.ShapeDtypeStruct((N_NODES, IN_D), jnp.float32),
    )(aggp, feat, root_pad, bias_row)


def _head_stats(z):
    def body(z1_ref, z2_ref, out_ref):
        j = pl.program_id(0)

        @pl.when(j == 0)
        def _():
            out_ref[...] = jnp.zeros((8, IN_D), jnp.float32)

        rows = lax.broadcasted_iota(jnp.int32, (TBLK, 1), 0) + j * TBLK
        m = (rows < N_TGT).astype(jnp.float32)
        z1 = z1_ref[...] * m
        z2 = z2_ref[...] * m
        s1 = jnp.sum(z1, axis=0, keepdims=True)
        q1 = jnp.sum(z1 * z1, axis=0, keepdims=True)
        s2 = jnp.sum(z2, axis=0, keepdims=True)
        q2 = jnp.sum(z2 * z2, axis=0, keepdims=True)
        zero4 = jnp.zeros((4, IN_D), jnp.float32)
        upd = jnp.concatenate([s1, q1, s2, q2, zero4], axis=0)
        out_ref[...] = out_ref[...] + upd

    return pl.pallas_call(
        body,
        grid=(TNB,),
        in_specs=[
            pl.BlockSpec((TBLK, IN_D), lambda j: (j, 0)),
            pl.BlockSpec((TBLK, IN_D), lambda j: (j + TNB, 0)),
        ],
        out_specs=pl.BlockSpec((8, IN_D), lambda j: (0, 0)),
        out_shape=jax.ShapeDtypeStruct((8, IN_D), jnp.float32),
    )(z, z)


def _head_apply(z, stats, ga, gb, ba, bb, wa, wb, lb_row):
    def body(z1_ref, z2_ref, st_ref, ga_ref, gb_ref, ba_ref, bb_ref,
             wa_ref, wb_ref, lb_ref, out_ref):
        cnt = float(N_TGT)
        mean1 = st_ref[0:1, :] / cnt
        msq1 = st_ref[1:2, :] / cnt
        mean2 = st_ref[2:3, :] / cnt
        msq2 = st_ref[3:4, :] / cnt
        inv1 = lax.rsqrt(msq1 - mean1 * mean1 + 1e-5)
        inv2 = lax.rsqrt(msq2 - mean2 * mean2 + 1e-5)
        sc1 = inv1 * ga_ref[...]
        sc2 = inv2 * gb_ref[...]
        sh1 = ba_ref[...] - mean1 * sc1
        sh2 = bb_ref[...] - mean2 * sc2
        wa = wa_ref[...]
        wb = wb_ref[...]
        bias_eff = (lax.dot_general(sh1, wa, _DN, preferred_element_type=jnp.float32)
                    + lax.dot_general(sh2, wb, _DN, preferred_element_type=jnp.float32)
                    + lb_ref[...])
        x1 = (z1_ref[...] * sc1).astype(jnp.bfloat16)
        x2 = (z2_ref[...] * sc2).astype(jnp.bfloat16)
        o = lax.dot_general(x1, wa.astype(jnp.bfloat16), _DN,
                            preferred_element_type=jnp.float32)
        o = o + lax.dot_general(x2, wb.astype(jnp.bfloat16), _DN,
                                preferred_element_type=jnp.float32)
        o = o + bias_eff
        out_ref[...] = 1.0 / (1.0 + jnp.exp(-o))

    return pl.pallas_call(
        body,
        grid=(TNB,),
        in_specs=[
            pl.BlockSpec((TBLK, IN_D), lambda j: (j, 0)),
            pl.BlockSpec((TBLK, IN_D), lambda j: (j + TNB, 0)),
            pl.BlockSpec((8, IN_D), lambda j: (0, 0)),
            pl.BlockSpec((1, IN_D), lambda j: (0, 0)),
            pl.BlockSpec((1, IN_D), lambda j: (0, 0)),
            pl.BlockSpec((1, IN_D), lambda j: (0, 0)),
            pl.BlockSpec((1, IN_D), lambda j: (0, 0)),
            pl.BlockSpec((IN_D, HID_D), lambda j: (0, 0)),
            pl.BlockSpec((IN_D, HID_D), lambda j: (0, 0)),
            pl.BlockSpec((1, HID_D), lambda j: (0, 0)),
        ],
        out_specs=pl.BlockSpec((TBLK, HID_D), lambda j: (j, 0)),
        out_shape=jax.ShapeDtypeStruct((T_PAD, HID_D), jnp.float32),
        compiler_params=pltpu.CompilerParams(
            dimension_semantics=("arbitrary",)),
    )(z, z, stats, ga, gb, ba, bb, wa, wb, lb_row)


def _prep_edge_weights(w1, b1, w2, b2, d_in):
    w1p = jnp.pad(w1, ((0, 0), (0, NNH_P - NNH)))
    b1p = jnp.pad(b1, (0, NNH_P - NNH))
    w2p = jnp.pad(w2, ((0, NNH_P - NNH), (0, 0)))
    w1b = jnp.repeat(w1p, HID_D, axis=1).astype(jnp.bfloat16)
    b1b = jnp.repeat(b1p, HID_D).reshape(1, WID)
    w2perm = (w2p.reshape(NNH_P, d_in, HID_D)
              .transpose(1, 0, 2).reshape(d_in, WID))
    w2perm = jnp.pad(w2perm, ((0, IN_D - d_in), (0, 0))).astype(jnp.bfloat16)
    b2r = jnp.pad(b2.reshape(d_in, HID_D),
                  ((0, IN_D - d_in), (0, 0))).astype(jnp.bfloat16)
    return w1b, b1b, w2perm, b2r


def _pad_lanes(v, d=IN_D):
    return jnp.pad(v.reshape(1, -1), ((0, 0), (0, d - v.shape[-1])))


def kernel(x, edge_index, edge_attr, target_edge_index,
           nn1_W1, nn1_b1, nn1_W2, nn1_b2, root1, bias1,
           nn2_W1, nn2_b1, nn2_W2, nn2_b2, root2, bias2,
           bn_g, bn_b, lin_W, lin_b):
    src = edge_index[0]
    dst = edge_index[1]
    e_fill = E_PAD - N_EDGES
    fill_idx = jnp.arange(e_fill, dtype=jnp.int32) % N_NODES
    src_p = jnp.concatenate([src, fill_idx])
    dst_p = jnp.concatenate([dst, jnp.full((e_fill,), N_NODES, jnp.int32)])
    ea_p = jnp.concatenate(
        [edge_attr, jnp.zeros((e_fill, ATTR_D), jnp.float32)])
    t_fill = jnp.arange(T_PAD - N_TGT, dtype=jnp.int32) % N_NODES
    t_fill2 = jnp.arange(T2_PAD - 2 * T_PAD, dtype=jnp.int32) % N_NODES
    tcat = jnp.concatenate(
        [target_edge_index[0], t_fill, target_edge_index[1], t_fill, t_fill2])
    src3d = src_p.reshape(NW, E_PAD // (NW * CH), CH)
    dst3d = dst_p.reshape(NW, E_PAD // (NW * CH), CH)
    t3d = tcat.reshape(NW, T2_PAD // (NW * CH), CH)
    zero_agg = jnp.zeros((N_PAD, IN_D), jnp.float32)

    w1b1, b1b1, w2p1, b2r1 = _prep_edge_weights(nn1_W1, nn1_b1, nn1_W2, nn1_b2, IN_D)
    w1b2, b1b2, w2p2, b2r2 = _prep_edge_weights(nn2_W1, nn2_b1, nn2_W2, nn2_b2, HID_D)
    s_sel = jnp.tile(jnp.eye(HID_D, dtype=jnp.bfloat16), (NNH_P, 1))
    root1b = root1.astype(jnp.bfloat16)
    root2b = jnp.pad(root2, ((0, IN_D - HID_D), (0, 0))).astype(jnp.bfloat16)

    xs = _sc_gather(x, src3d, E_PAD, E_PAD // (NW * CH))
    msg1 = _edge_messages(xs, ea_p, w1b1, b1b1, w2p1, s_sel, b2r1, blk=1024)
    agg1 = _sc_scatter_add(msg1, dst3d, zero_agg)
    h = _node_update(agg1, x, root1b, bias1.reshape(1, HID_D), relu=True)
    hs = _sc_gather(h, src3d, E_PAD, E_PAD // (NW * CH))
    msg2 = _edge_messages(hs, ea_p, w1b2, b1b2, w2p2, s_sel, b2r2, blk=1024)
    agg2 = _sc_scatter_add(msg2, dst3d, zero_agg)
    h2 = _node_update(agg2, h, root2b, bias2.reshape(1, HID_D), relu=False)
    z = _sc_gather(h2, t3d, T2_PAD, T2_PAD // (NW * CH))
    stats = _head_stats(z)
    out = _head_apply(z, stats,
                      _pad_lanes(bn_g[:HID_D]), _pad_lanes(bn_g[HID_D:]),
                      _pad_lanes(bn_b[:HID_D]), _pad_lanes(bn_b[HID_D:]),
                      jnp.pad(lin_W[:HID_D], ((0, IN_D - HID_D), (0, 0))),
                      jnp.pad(lin_W[HID_D:], ((0, IN_D - HID_D), (0, 0))),
                      lin_b.reshape(1, HID_D))
    return out[:N_TGT]


# speedup vs baseline: 1.7251x; 1.0319x over previous
import functools

import jax
import jax.numpy as jnp
from jax import lax
from jax.experimental import pallas as pl
from jax.experimental.pallas import tpu as pltpu
from jax.experimental.pallas import tpu_sc as plsc

N_NODES = 10000
N_EDGES = 160000
N_TGT = 50000
IN_D = 128
HID_D = 16
ATTR_D = 16
NNH = 100
NNH_P = 104
WID = NNH_P * 16

NC, NS = 2, 16
NW = NC * NS
CH = 128

E_PAD = 163840
T_PAD = 53248
T2_PAD = 114688
N_PAD = 10112
TBLK = 4096
TNB = T_PAD // TBLK


def _vmesh():
    return plsc.VectorSubcoreMesh(core_axis_name="c", subcore_axis_name="s")


def _sc_gather(table, idx3d, out_rows, n_ch):
    per_w = out_rows // NW
    NB = 4

    @functools.partial(
        pl.kernel,
        out_type=jax.ShapeDtypeStruct((out_rows, IN_D), jnp.float32),
        mesh=_vmesh(),
        scratch_types=[
            pltpu.VMEM((n_ch, CH), jnp.int32),
            pltpu.VMEM((NB, CH, IN_D), jnp.float32),
            pltpu.SemaphoreType.DMA,
            pltpu.SemaphoreType.DMA,
            pltpu.SemaphoreType.DMA,
            pltpu.SemaphoreType.DMA,
            pltpu.SemaphoreType.DMA,
            pltpu.SemaphoreType.DMA,
            pltpu.SemaphoreType.DMA,
            pltpu.SemaphoreType.DMA,
        ],
    )
    def k(table_hbm, idx_hbm, out_hbm, idx_v, rows_v, *sems):
        gsem = sems[:NB]
        wsem = sems[NB:]
        wid = lax.axis_index("s") * NC + lax.axis_index("c")
        base = wid * per_w
        pltpu.sync_copy(idx_hbm.at[wid], idx_v)

        def g_desc(jj, b):
            return pltpu.make_async_copy(
                table_hbm.at[idx_v.at[jj]], rows_v.at[b], gsem[b])

        def w_desc(jj, b):
            return pltpu.make_async_copy(
                rows_v.at[b], out_hbm.at[pl.ds(base + jj * CH, CH)], wsem[b])

        for b in range(NB):
            g_desc(b, b).start()

        @pl.loop(0, n_ch, step=NB)
        def _(j):
            for b in range(NB):
                g_desc(j + b, b).wait()
                w_desc(j + b, b).start()
            for b in range(NB):
                w_desc(j + b, b).wait()

                @pl.when(j + NB + b < n_ch)
                def _():
                    g_desc(j + NB + b, b).start()

    return k(table, idx3d)


def _sc_scatter_add(msg, dst3d, zero_init):
    rows_per_tile = N_PAD // NS
    e_half = E_PAD // NC
    per_w = e_half // NS
    n_ch = per_w // CH
    NB = 2

    @functools.partial(
        pl.kernel,
        out_type=jax.ShapeDtypeStruct((NC, N_PAD, IN_D), jnp.float32),
        mesh=_vmesh(),
        scratch_types=[
            pltpu.VMEM((n_ch, CH), jnp.int32),
            pltpu.VMEM((NB, CH, IN_D), jnp.float32),
            pltpu.VMEM_SHARED((N_PAD, IN_D), jnp.float32),
            pltpu.SemaphoreType.DMA,
            pltpu.SemaphoreType.DMA,
            pltpu.SemaphoreType.DMA,
            pltpu.SemaphoreType.DMA,
        ],
    )
    def k(msg_hbm, dst_hbm, zero_hbm, out_hbm, idx_v, val_v, agg_sh, *sems):
        lsem = sems[:NB]
        ssem = sems[NB:]
        c = lax.axis_index("c")
        s = lax.axis_index("s")
        wid = s * NC + c
        r0 = s * rows_per_tile
        pltpu.sync_copy(dst_hbm.at[wid], idx_v)
        pltpu.sync_copy(zero_hbm.at[pl.ds(r0, rows_per_tile)],
                        agg_sh.at[pl.ds(r0, rows_per_tile)])
        plsc.subcore_barrier()
        base = wid * per_w

        def l_desc(jj, b):
            return pltpu.make_async_copy(
                msg_hbm.at[pl.ds(base + jj * CH, CH)], val_v.at[b], lsem[b])

        def s_desc(jj, b):
            return pltpu.make_async_copy(
                val_v.at[b], agg_sh.at[idx_v.at[jj]], ssem[b])

        for b in range(NB):
            l_desc(b, b).start()

        @pl.loop(0, n_ch, step=NB)
        def _(j):
            for b in range(NB):
                l_desc(j + b, b).wait()
                s_desc(j + b, b).start(add=True)
            for b in range(NB):
                s_desc(j + b, b).wait()

                @pl.when(j + NB + b < n_ch)
                def _():
                    l_desc(j + NB + b, b).start()

        plsc.subcore_barrier()
        pltpu.sync_copy(agg_sh.at[pl.ds(r0, rows_per_tile)],
                        out_hbm.at[c, pl.ds(r0, rows_per_tile)])

    return k(msg, dst3d, zero_init)


_DN = (((1,), (0,)), ((), ()))


def _edge_messages(feat, ea, w1b, b1b, w2p, s_sel, b2r, blk):
    def body(feat_ref, ea_ref, w1b_ref, b1b_ref, w2p_ref, s_ref, b2r_ref, out_ref):
        ea_b = ea_ref[...].astype(jnp.bfloat16)
        pre = lax.dot_general(ea_b, w1b_ref[...], _DN,
                              preferred_element_type=jnp.float32) + b1b_ref[...]
        h2 = jnp.maximum(pre, 0.0)
        xb = feat_ref[...].astype(jnp.bfloat16)
        g = lax.dot_general(xb, w2p_ref[...], _DN,
                            preferred_element_type=jnp.float32)
        p = (h2 * g).astype(jnp.bfloat16)
        m = lax.dot_general(p, s_ref[...], _DN, preferred_element_type=jnp.float32)
        m = m + lax.dot_general(xb, b2r_ref[...], _DN,
                                preferred_element_type=jnp.float32)
        out_ref[...] = jnp.concatenate(
            [m, jnp.zeros((m.shape[0], IN_D - HID_D), jnp.float32)], axis=1)

    return pl.pallas_call(
        body,
        grid=(E_PAD // blk,),
        in_specs=[
            pl.BlockSpec((blk, IN_D), lambda i: (i, 0)),
            pl.BlockSpec((blk, ATTR_D), lambda i: (i, 0)),
            pl.BlockSpec((ATTR_D, WID), lambda i: (0, 0)),
            pl.BlockSpec((1, WID), lambda i: (0, 0)),
            pl.BlockSpec((IN_D, WID), lambda i: (0, 0)),
            pl.BlockSpec((WID, HID_D), lambda i: (0, 0)),
            pl.BlockSpec((IN_D, HID_D), lambda i: (0, 0)),
        ],
        out_specs=pl.BlockSpec((blk, IN_D), lambda i: (i, 0)),
        out_shape=jax.ShapeDtypeStruct((E_PAD, IN_D), jnp.float32),
        compiler_params=pltpu.CompilerParams(
            dimension_semantics=("parallel",)),
    )(feat, ea, w1b, b1b, w2p, s_sel, b2r)


def _node_update(aggp, feat, root_pad, bias_row, relu):
    def body(agg_ref, feat_ref, root_ref, bias_ref, out_ref):
        a = (agg_ref[0, :N_NODES, :HID_D]
             + agg_ref[1, :N_NODES, :HID_D])
        fb = feat_ref[...].astype(jnp.bfloat16)
        r = lax.dot_general(fb, root_ref[...], _DN,
                            preferred_element_type=jnp.float32)
        o = a + r + bias_ref[...]
        if relu:
            o = jnp.maximum(o, 0.0)
        out_ref[...] = jnp.concatenate(
            [o, jnp.zeros((N_NODES, IN_D - HID_D), jnp.float32)], axis=1)

    return pl.pallas_call(
        body,
        in_specs=[
            pl.BlockSpec((NC, N_PAD, IN_D), lambda: (0, 0, 0)),
            pl.BlockSpec((N_NODES, IN_D), lambda: (0, 0)),
            pl.BlockSpec((IN_D, HID_D), lambda: (0, 0)),
            pl.BlockSpec((1, HID_D), lambda: (0, 0)),
        ],
        out_specs=pl.BlockSpec((N_NODES, IN_D), lambda: (0, 0)),
        out_shape=jax.ShapeDtypeStruct((N_NODES, IN_D), jnp.float32),
    )(aggp, feat, root_pad, bias_row)


def _head_stats(z):
    def body(z1_ref, z2_ref, out_ref):
        j = pl.program_id(0)

        @pl.when(j == 0)
        def _():
            out_ref[...] = jnp.zeros((8, IN_D), jnp.float32)

        rows = lax.broadcasted_iota(jnp.int32, (TBLK, 1), 0) + j * TBLK
        m = (rows < N_TGT).astype(jnp.float32)
        z1 = z1_ref[...] * m
        z2 = z2_ref[...] * m
        s1 = jnp.sum(z1, axis=0, keepdims=True)
        q1 = jnp.sum(z1 * z1, axis=0, keepdims=True)
        s2 = jnp.sum(z2, axis=0, keepdims=True)
        q2 = jnp.sum(z2 * z2, axis=0, keepdims=True)
        zero4 = jnp.zeros((4, IN_D), jnp.float32)
        upd = jnp.concatenate([s1, q1, s2, q2, zero4], axis=0)
        out_ref[...] = out_ref[...] + upd

    return pl.pallas_call(
        body,
        grid=(TNB,),
        in_specs=[
            pl.BlockSpec((TBLK, IN_D), lambda j: (j, 0)),
            pl.BlockSpec((TBLK, IN_D), lambda j: (j + TNB, 0)),
        ],
        out_specs=pl.BlockSpec((8, IN_D), lambda j: (0, 0)),
        out_shape=jax.ShapeDtypeStruct((8, IN_D), jnp.float32),
    )(z, z)


def _head_apply(z, stats, ga, gb, ba, bb, wa, wb, lb_row):
    def body(z1_ref, z2_ref, st_ref, ga_ref, gb_ref, ba_ref, bb_ref,
             wa_ref, wb_ref, lb_ref, out_ref):
        cnt = float(N_TGT)
        mean1 = st_ref[0:1, :] / cnt
        msq1 = st_ref[1:2, :] / cnt
        mean2 = st_ref[2:3, :] / cnt
        msq2 = st_ref[3:4, :] / cnt
        inv1 = lax.rsqrt(msq1 - mean1 * mean1 + 1e-5)
        inv2 = lax.rsqrt(msq2 - mean2 * mean2 + 1e-5)
        sc1 = inv1 * ga_ref[...]
        sc2 = inv2 * gb_ref[...]
        sh1 = ba_ref[...] - mean1 * sc1
        sh2 = bb_ref[...] - mean2 * sc2
        wa = wa_ref[...]
        wb = wb_ref[...]
        bias_eff = (lax.dot_general(sh1, wa, _DN, preferred_element_type=jnp.float32)
                    + lax.dot_general(sh2, wb, _DN, preferred_element_type=jnp.float32)
                    + lb_ref[...])
        x1 = (z1_ref[...] * sc1).astype(jnp.bfloat16)
        x2 = (z2_ref[...] * sc2).astype(jnp.bfloat16)
        o = lax.dot_general(x1, wa.astype(jnp.bfloat16), _DN,
                            preferred_element_type=jnp.float32)
        o = o + lax.dot_general(x2, wb.astype(jnp.bfloat16), _DN,
                                preferred_element_type=jnp.float32)
        o = o + bias_eff
        out_ref[...] = 1.0 / (1.0 + jnp.exp(-o))

    return pl.pallas_call(
        body,
        grid=(TNB,),
        in_specs=[
            pl.BlockSpec((TBLK, IN_D), lambda j: (j, 0)),
            pl.BlockSpec((TBLK, IN_D), lambda j: (j + TNB, 0)),
            pl.BlockSpec((8, IN_D), lambda j: (0, 0)),
            pl.BlockSpec((1, IN_D), lambda j: (0, 0)),
            pl.BlockSpec((1, IN_D), lambda j: (0, 0)),
            pl.BlockSpec((1, IN_D), lambda j: (0, 0)),
            pl.BlockSpec((1, IN_D), lambda j: (0, 0)),
            pl.BlockSpec((IN_D, HID_D), lambda j: (0, 0)),
            pl.BlockSpec((IN_D, HID_D), lambda j: (0, 0)),
            pl.BlockSpec((1, HID_D), lambda j: (0, 0)),
        ],
        out_specs=pl.BlockSpec((TBLK, HID_D), lambda j: (j, 0)),
        out_shape=jax.ShapeDtypeStruct((T_PAD, HID_D), jnp.float32),
        compiler_params=pltpu.CompilerParams(
            dimension_semantics=("arbitrary",)),
    )(z, z, stats, ga, gb, ba, bb, wa, wb, lb_row)


def _prep_edge_weights(w1, b1, w2, b2, d_in):
    w1p = jnp.pad(w1, ((0, 0), (0, NNH_P - NNH)))
    b1p = jnp.pad(b1, (0, NNH_P - NNH))
    w2p = jnp.pad(w2, ((0, NNH_P - NNH), (0, 0)))
    w1b = jnp.repeat(w1p, HID_D, axis=1).astype(jnp.bfloat16)
    b1b = jnp.repeat(b1p, HID_D).reshape(1, WID)
    w2perm = (w2p.reshape(NNH_P, d_in, HID_D)
              .transpose(1, 0, 2).reshape(d_in, WID))
    w2perm = jnp.pad(w2perm, ((0, IN_D - d_in), (0, 0))).astype(jnp.bfloat16)
    b2r = jnp.pad(b2.reshape(d_in, HID_D),
                  ((0, IN_D - d_in), (0, 0))).astype(jnp.bfloat16)
    return w1b, b1b, w2perm, b2r


def _pad_lanes(v, d=IN_D):
    return jnp.pad(v.reshape(1, -1), ((0, 0), (0, d - v.shape[-1])))


def kernel(x, edge_index, edge_attr, target_edge_index,
           nn1_W1, nn1_b1, nn1_W2, nn1_b2, root1, bias1,
           nn2_W1, nn2_b1, nn2_W2, nn2_b2, root2, bias2,
           bn_g, bn_b, lin_W, lin_b):
    src = edge_index[0]
    dst = edge_index[1]
    e_fill = E_PAD - N_EDGES
    fill_idx = jnp.arange(e_fill, dtype=jnp.int32) % N_NODES
    src_p = jnp.concatenate([src, fill_idx])
    dst_p = jnp.concatenate([dst, jnp.full((e_fill,), N_NODES, jnp.int32)])
    ea_p = jnp.concatenate(
        [edge_attr, jnp.zeros((e_fill, ATTR_D), jnp.float32)])
    t_fill = jnp.arange(T_PAD - N_TGT, dtype=jnp.int32) % N_NODES
    t_fill2 = jnp.arange(T2_PAD - 2 * T_PAD, dtype=jnp.int32) % N_NODES
    tcat = jnp.concatenate(
        [target_edge_index[0], t_fill, target_edge_index[1], t_fill, t_fill2])
    src3d = src_p.reshape(NW, E_PAD // (NW * CH), CH)
    dst3d = dst_p.reshape(NW, E_PAD // (NW * CH), CH)
    t3d = tcat.reshape(NW, T2_PAD // (NW * CH), CH)
    zero_agg = jnp.zeros((N_PAD, IN_D), jnp.float32)

    w1b1, b1b1, w2p1, b2r1 = _prep_edge_weights(nn1_W1, nn1_b1, nn1_W2, nn1_b2, IN_D)
    w1b2, b1b2, w2p2, b2r2 = _prep_edge_weights(nn2_W1, nn2_b1, nn2_W2, nn2_b2, HID_D)
    s_sel = jnp.tile(jnp.eye(HID_D, dtype=jnp.bfloat16), (NNH_P, 1))
    root1b = root1.astype(jnp.bfloat16)
    root2b = jnp.pad(root2, ((0, IN_D - HID_D), (0, 0))).astype(jnp.bfloat16)

    xs = _sc_gather(x, src3d, E_PAD, E_PAD // (NW * CH))
    msg1 = _edge_messages(xs, ea_p, w1b1, b1b1, w2p1, s_sel, b2r1, blk=2048)
    agg1 = _sc_scatter_add(msg1, dst3d, zero_agg)
    h = _node_update(agg1, x, root1b, bias1.reshape(1, HID_D), relu=True)
    hs = _sc_gather(h, src3d, E_PAD, E_PAD // (NW * CH))
    msg2 = _edge_messages(hs, ea_p, w1b2, b1b2, w2p2, s_sel, b2r2, blk=2048)
    agg2 = _sc_scatter_add(msg2, dst3d, zero_agg)
    h2 = _node_update(agg2, h, root2b, bias2.reshape(1, HID_D), relu=False)
    z = _sc_gather(h2, t3d, T2_PAD, T2_PAD // (NW * CH))
    stats = _head_stats(z)
    out = _head_apply(z, stats,
                      _pad_lanes(bn_g[:HID_D]), _pad_lanes(bn_g[HID_D:]),
                      _pad_lanes(bn_b[:HID_D]), _pad_lanes(bn_b[HID_D:]),
                      jnp.pad(lin_W[:HID_D], ((0, IN_D - HID_D), (0, 0))),
                      jnp.pad(lin_W[HID_D:], ((0, IN_D - HID_D), (0, 0))),
                      lin_b.reshape(1, HID_D))
    return out[:N_TGT]


# speedup vs baseline: 1.8532x; 1.0742x over previous
import functools

import jax
import jax.numpy as jnp
from jax import lax
from jax.experimental import pallas as pl
from jax.experimental.pallas import tpu as pltpu
from jax.experimental.pallas import tpu_sc as plsc

N_NODES = 10000
N_EDGES = 160000
N_TGT = 50000
IN_D = 128
HID_D = 16
ATTR_D = 16
NNH = 100
NNH_P = 104
WID = NNH_P * 16

NC, NS = 2, 16
NW = NC * NS
CH = 128

E_PAD = 163840
T_PAD = 53248
T2_PAD = 114688
N_PAD = 10112
TBLK = 4096
TNB = T_PAD // TBLK


def _vmesh():
    return plsc.VectorSubcoreMesh(core_axis_name="c", subcore_axis_name="s")


def _sc_gather(table, idx3d, out_rows, n_ch):
    per_w = out_rows // NW
    NB = 4

    @functools.partial(
        pl.kernel,
        out_type=jax.ShapeDtypeStruct((out_rows, IN_D), jnp.float32),
        mesh=_vmesh(),
        scratch_types=[
            pltpu.VMEM((n_ch, CH), jnp.int32),
            pltpu.VMEM((NB, CH, IN_D), jnp.float32),
            pltpu.SemaphoreType.DMA,
            pltpu.SemaphoreType.DMA,
            pltpu.SemaphoreType.DMA,
            pltpu.SemaphoreType.DMA,
            pltpu.SemaphoreType.DMA,
            pltpu.SemaphoreType.DMA,
            pltpu.SemaphoreType.DMA,
            pltpu.SemaphoreType.DMA,
        ],
    )
    def k(table_hbm, idx_hbm, out_hbm, idx_v, rows_v, *sems):
        gsem = sems[:NB]
        wsem = sems[NB:]
        wid = lax.axis_index("s") * NC + lax.axis_index("c")
        base = wid * per_w
        pltpu.sync_copy(idx_hbm.at[wid], idx_v)

        def g_desc(jj, b):
            return pltpu.make_async_copy(
                table_hbm.at[idx_v.at[jj]], rows_v.at[b], gsem[b])

        def w_desc(jj, b):
            return pltpu.make_async_copy(
                rows_v.at[b], out_hbm.at[pl.ds(base + jj * CH, CH)], wsem[b])

        for b in range(NB):
            g_desc(b, b).start()

        @pl.loop(0, n_ch, step=NB)
        def _(j):
            for b in range(NB):
                g_desc(j + b, b).wait()
                w_desc(j + b, b).start()
            for b in range(NB):
                w_desc(j + b, b).wait()

                @pl.when(j + NB + b < n_ch)
                def _():
                    g_desc(j + NB + b, b).start()

    return k(table, idx3d)


def _sc_scatter_add(msg, dst3d, init):
    e_rows = msg.shape[0]
    rows_per_tile = N_PAD // NS
    per_w = e_rows // NW
    n_ch = per_w // CH
    NB = 2

    @functools.partial(
        pl.kernel,
        out_type=jax.ShapeDtypeStruct((NC, N_PAD, IN_D), jnp.float32),
        mesh=_vmesh(),
        scratch_types=[
            pltpu.VMEM((n_ch, CH), jnp.int32),
            pltpu.VMEM((NB, CH, IN_D), jnp.float32),
            pltpu.VMEM_SHARED((N_PAD, IN_D), jnp.float32),
            pltpu.SemaphoreType.DMA,
            pltpu.SemaphoreType.DMA,
            pltpu.SemaphoreType.DMA,
            pltpu.SemaphoreType.DMA,
        ],
    )
    def k(msg_hbm, dst_hbm, init_hbm, out_hbm, idx_v, val_v, agg_sh, *sems):
        lsem = sems[:NB]
        ssem = sems[NB:]
        c = lax.axis_index("c")
        s = lax.axis_index("s")
        wid = s * NC + c
        r0 = s * rows_per_tile
        pltpu.sync_copy(dst_hbm.at[wid], idx_v)
        pltpu.sync_copy(init_hbm.at[c, pl.ds(r0, rows_per_tile)],
                        agg_sh.at[pl.ds(r0, rows_per_tile)])
        plsc.subcore_barrier()
        base = wid * per_w

        def l_desc(jj, b):
            return pltpu.make_async_copy(
                msg_hbm.at[pl.ds(base + jj * CH, CH)], val_v.at[b], lsem[b])

        def s_desc(jj, b):
            return pltpu.make_async_copy(
                val_v.at[b], agg_sh.at[idx_v.at[jj]], ssem[b])

        for b in range(NB):
            l_desc(b, b).start()

        @pl.loop(0, n_ch, step=NB)
        def _(j):
            for b in range(NB):
                l_desc(j + b, b).wait()
                s_desc(j + b, b).start(add=True)
            for b in range(NB):
                s_desc(j + b, b).wait()

                @pl.when(j + NB + b < n_ch)
                def _():
                    l_desc(j + NB + b, b).start()

        plsc.subcore_barrier()
        pltpu.sync_copy(agg_sh.at[pl.ds(r0, rows_per_tile)],
                        out_hbm.at[c, pl.ds(r0, rows_per_tile)])

    return k(msg, dst3d, init)


_DN = (((1,), (0,)), ((), ()))


def _edge_messages(feat, ea, w1b, b1b, w2p, s_sel, b2r, blk):
    rows = feat.shape[0]

    def body(feat_ref, ea_ref, w1b_ref, b1b_ref, w2p_ref, s_ref, b2r_ref, out_ref):
        ea_b = ea_ref[...].astype(jnp.bfloat16)
        pre = lax.dot_general(ea_b, w1b_ref[...], _DN,
                              preferred_element_type=jnp.float32) + b1b_ref[...]
        h2 = jnp.maximum(pre, 0.0)
        xb = feat_ref[...].astype(jnp.bfloat16)
        g = lax.dot_general(xb, w2p_ref[...], _DN,
                            preferred_element_type=jnp.float32)
        p = (h2 * g).astype(jnp.bfloat16)
        m = lax.dot_general(p, s_ref[...], _DN, preferred_element_type=jnp.float32)
        m = m + lax.dot_general(xb, b2r_ref[...], _DN,
                                preferred_element_type=jnp.float32)
        out_ref[...] = jnp.concatenate(
            [m, jnp.zeros((m.shape[0], IN_D - HID_D), jnp.float32)], axis=1)

    return pl.pallas_call(
        body,
        grid=(rows // blk,),
        in_specs=[
            pl.BlockSpec((blk, IN_D), lambda i: (i, 0)),
            pl.BlockSpec((blk, ATTR_D), lambda i: (i, 0)),
            pl.BlockSpec((ATTR_D, WID), lambda i: (0, 0)),
            pl.BlockSpec((1, WID), lambda i: (0, 0)),
            pl.BlockSpec((IN_D, WID), lambda i: (0, 0)),
            pl.BlockSpec((WID, HID_D), lambda i: (0, 0)),
            pl.BlockSpec((IN_D, HID_D), lambda i: (0, 0)),
        ],
        out_specs=pl.BlockSpec((blk, IN_D), lambda i: (i, 0)),
        out_shape=jax.ShapeDtypeStruct((rows, IN_D), jnp.float32),
        compiler_params=pltpu.CompilerParams(
            dimension_semantics=("parallel",)),
    )(feat, ea, w1b, b1b, w2p, s_sel, b2r)


def _node_update(aggp, feat, root_pad, bias_row, relu):
    def body(agg_ref, feat_ref, root_ref, bias_ref, out_ref):
        a = (agg_ref[0, :N_NODES, :HID_D]
             + agg_ref[1, :N_NODES, :HID_D])
        fb = feat_ref[...].astype(jnp.bfloat16)
        r = lax.dot_general(fb, root_ref[...], _DN,
                            preferred_element_type=jnp.float32)
        o = a + r + bias_ref[...]
        if relu:
            o = jnp.maximum(o, 0.0)
        out_ref[...] = jnp.concatenate(
            [o, jnp.zeros((N_NODES, IN_D - HID_D), jnp.float32)], axis=1)

    return pl.pallas_call(
        body,
        in_specs=[
            pl.BlockSpec((NC, N_PAD, IN_D), lambda: (0, 0, 0)),
            pl.BlockSpec((N_NODES, IN_D), lambda: (0, 0)),
            pl.BlockSpec((IN_D, HID_D), lambda: (0, 0)),
            pl.BlockSpec((1, HID_D), lambda: (0, 0)),
        ],
        out_specs=pl.BlockSpec((N_NODES, IN_D), lambda: (0, 0)),
        out_shape=jax.ShapeDtypeStruct((N_NODES, IN_D), jnp.float32),
    )(aggp, feat, root_pad, bias_row)


def _head_stats(z):
    def body(z1_ref, z2_ref, out_ref):
        j = pl.program_id(0)

        @pl.when(j == 0)
        def _():
            out_ref[...] = jnp.zeros((8, IN_D), jnp.float32)

        rows = lax.broadcasted_iota(jnp.int32, (TBLK, 1), 0) + j * TBLK
        m = (rows < N_TGT).astype(jnp.float32)
        z1 = z1_ref[...] * m
        z2 = z2_ref[...] * m
        s1 = jnp.sum(z1, axis=0, keepdims=True)
        q1 = jnp.sum(z1 * z1, axis=0, keepdims=True)
        s2 = jnp.sum(z2, axis=0, keepdims=True)
        q2 = jnp.sum(z2 * z2, axis=0, keepdims=True)
        zero4 = jnp.zeros((4, IN_D), jnp.float32)
        upd = jnp.concatenate([s1, q1, s2, q2, zero4], axis=0)
        out_ref[...] = out_ref[...] + upd

    return pl.pallas_call(
        body,
        grid=(TNB,),
        in_specs=[
            pl.BlockSpec((TBLK, IN_D), lambda j: (j, 0)),
            pl.BlockSpec((TBLK, IN_D), lambda j: (j + TNB, 0)),
        ],
        out_specs=pl.BlockSpec((8, IN_D), lambda j: (0, 0)),
        out_shape=jax.ShapeDtypeStruct((8, IN_D), jnp.float32),
    )(z, z)


def _head_apply(z, stats, ga, gb, ba, bb, wa, wb, lb_row):
    def body(z1_ref, z2_ref, st_ref, ga_ref, gb_ref, ba_ref, bb_ref,
             wa_ref, wb_ref, lb_ref, out_ref):
        cnt = float(N_TGT)
        mean1 = st_ref[0:1, :] / cnt
        msq1 = st_ref[1:2, :] / cnt
        mean2 = st_ref[2:3, :] / cnt
        msq2 = st_ref[3:4, :] / cnt
        inv1 = lax.rsqrt(msq1 - mean1 * mean1 + 1e-5)
        inv2 = lax.rsqrt(msq2 - mean2 * mean2 + 1e-5)
        sc1 = inv1 * ga_ref[...]
        sc2 = inv2 * gb_ref[...]
        sh1 = ba_ref[...] - mean1 * sc1
        sh2 = bb_ref[...] - mean2 * sc2
        wa = wa_ref[...]
        wb = wb_ref[...]
        bias_eff = (lax.dot_general(sh1, wa, _DN, preferred_element_type=jnp.float32)
                    + lax.dot_general(sh2, wb, _DN, preferred_element_type=jnp.float32)
                    + lb_ref[...])
        x1 = (z1_ref[...] * sc1).astype(jnp.bfloat16)
        x2 = (z2_ref[...] * sc2).astype(jnp.bfloat16)
        o = lax.dot_general(x1, wa.astype(jnp.bfloat16), _DN,
                            preferred_element_type=jnp.float32)
        o = o + lax.dot_general(x2, wb.astype(jnp.bfloat16), _DN,
                                preferred_element_type=jnp.float32)
        o = o + bias_eff
        out_ref[...] = 1.0 / (1.0 + jnp.exp(-o))

    return pl.pallas_call(
        body,
        grid=(TNB,),
        in_specs=[
            pl.BlockSpec((TBLK, IN_D), lambda j: (j, 0)),
            pl.BlockSpec((TBLK, IN_D), lambda j: (j + TNB, 0)),
            pl.BlockSpec((8, IN_D), lambda j: (0, 0)),
            pl.BlockSpec((1, IN_D), lambda j: (0, 0)),
            pl.BlockSpec((1, IN_D), lambda j: (0, 0)),
            pl.BlockSpec((1, IN_D), lambda j: (0, 0)),
            pl.BlockSpec((1, IN_D), lambda j: (0, 0)),
            pl.BlockSpec((IN_D, HID_D), lambda j: (0, 0)),
            pl.BlockSpec((IN_D, HID_D), lambda j: (0, 0)),
            pl.BlockSpec((1, HID_D), lambda j: (0, 0)),
        ],
        out_specs=pl.BlockSpec((TBLK, HID_D), lambda j: (j, 0)),
        out_shape=jax.ShapeDtypeStruct((T_PAD, HID_D), jnp.float32),
        compiler_params=pltpu.CompilerParams(
            dimension_semantics=("arbitrary",)),
    )(z, z, stats, ga, gb, ba, bb, wa, wb, lb_row)


def _prep_edge_weights(w1, b1, w2, b2, d_in):
    w1p = jnp.pad(w1, ((0, 0), (0, NNH_P - NNH)))
    b1p = jnp.pad(b1, (0, NNH_P - NNH))
    w2p = jnp.pad(w2, ((0, NNH_P - NNH), (0, 0)))
    w1b = jnp.repeat(w1p, HID_D, axis=1).astype(jnp.bfloat16)
    b1b = jnp.repeat(b1p, HID_D).reshape(1, WID)
    w2perm = (w2p.reshape(NNH_P, d_in, HID_D)
              .transpose(1, 0, 2).reshape(d_in, WID))
    w2perm = jnp.pad(w2perm, ((0, IN_D - d_in), (0, 0))).astype(jnp.bfloat16)
    b2r = jnp.pad(b2.reshape(d_in, HID_D),
                  ((0, IN_D - d_in), (0, 0))).astype(jnp.bfloat16)
    return w1b, b1b, w2perm, b2r


def _pad_lanes(v, d=IN_D):
    return jnp.pad(v.reshape(1, -1), ((0, 0), (0, d - v.shape[-1])))


def kernel(x, edge_index, edge_attr, target_edge_index,
           nn1_W1, nn1_b1, nn1_W2, nn1_b2, root1, bias1,
           nn2_W1, nn2_b1, nn2_W2, nn2_b2, root2, bias2,
           bn_g, bn_b, lin_W, lin_b):
    src = edge_index[0]
    dst = edge_index[1]
    e_fill = E_PAD - N_EDGES
    fill_idx = jnp.arange(e_fill, dtype=jnp.int32) % N_NODES
    src_p = jnp.concatenate([src, fill_idx])
    dst_p = jnp.concatenate([dst, jnp.full((e_fill,), N_NODES, jnp.int32)])
    ea_p = jnp.concatenate(
        [edge_attr, jnp.zeros((e_fill, ATTR_D), jnp.float32)])
    t_fill = jnp.arange(T_PAD - N_TGT, dtype=jnp.int32) % N_NODES
    t_fill2 = jnp.arange(T2_PAD - 2 * T_PAD, dtype=jnp.int32) % N_NODES
    tcat = jnp.concatenate(
        [target_edge_index[0], t_fill, target_edge_index[1], t_fill, t_fill2])
    eh = E_PAD // 2
    nch_h = eh // (NW * CH)
    src3d = [src_p[:eh].reshape(NW, nch_h, CH), src_p[eh:].reshape(NW, nch_h, CH)]
    dst3d = [dst_p[:eh].reshape(NW, nch_h, CH), dst_p[eh:].reshape(NW, nch_h, CH)]
    ea_c = [ea_p[:eh], ea_p[eh:]]
    t3d = tcat.reshape(NW, T2_PAD // (NW * CH), CH)
    zero_agg = jnp.zeros((NC, N_PAD, IN_D), jnp.float32)

    w1b1, b1b1, w2p1, b2r1 = _prep_edge_weights(nn1_W1, nn1_b1, nn1_W2, nn1_b2, IN_D)
    w1b2, b1b2, w2p2, b2r2 = _prep_edge_weights(nn2_W1, nn2_b1, nn2_W2, nn2_b2, HID_D)
    s_sel = jnp.tile(jnp.eye(HID_D, dtype=jnp.bfloat16), (NNH_P, 1))
    root1b = root1.astype(jnp.bfloat16)
    root2b = jnp.pad(root2, ((0, IN_D - HID_D), (0, 0))).astype(jnp.bfloat16)

    xs0 = _sc_gather(x, src3d[0], eh, nch_h)
    xs1 = _sc_gather(x, src3d[1], eh, nch_h)
    m10 = _edge_messages(xs0, ea_c[0], w1b1, b1b1, w2p1, s_sel, b2r1, blk=2048)
    m11 = _edge_messages(xs1, ea_c[1], w1b1, b1b1, w2p1, s_sel, b2r1, blk=2048)
    a10 = _sc_scatter_add(m10, dst3d[0], zero_agg)
    agg1 = _sc_scatter_add(m11, dst3d[1], a10)
    h = _node_update(agg1, x, root1b, bias1.reshape(1, HID_D), relu=True)
    hs0 = _sc_gather(h, src3d[0], eh, nch_h)
    hs1 = _sc_gather(h, src3d[1], eh, nch_h)
    m20 = _edge_messages(hs0, ea_c[0], w1b2, b1b2, w2p2, s_sel, b2r2, blk=2048)
    m21 = _edge_messages(hs1, ea_c[1], w1b2, b1b2, w2p2, s_sel, b2r2, blk=2048)
    a20 = _sc_scatter_add(m20, dst3d[0], zero_agg)
    agg2 = _sc_scatter_add(m21, dst3d[1], a20)
    h2 = _node_update(agg2, h, root2b, bias2.reshape(1, HID_D), relu=False)
    z = _sc_gather(h2, t3d, T2_PAD, T2_PAD // (NW * CH))
    stats = _head_stats(z)
    out = _head_apply(z, stats,
                      _pad_lanes(bn_g[:HID_D]), _pad_lanes(bn_g[HID_D:]),
                      _pad_lanes(bn_b[:HID_D]), _pad_lanes(bn_b[HID_D:]),
                      jnp.pad(lin_W[:HID_D], ((0, IN_D - HID_D), (0, 0))),
                      jnp.pad(lin_W[HID_D:], ((0, IN_D - HID_D), (0, 0))),
                      lin_b.reshape(1, HID_D))
    return out[:N_TGT]
